# baseline (speedup 1.0000x reference)
import functools

import jax
import jax.numpy as jnp
from jax import lax
from jax.experimental import pallas as pl
from jax.experimental.pallas import tpu as pltpu
from jax.experimental.pallas import tpu_sc as plsc

TOTAL = 32768
NSEG = 16
L = 16
NC = 2
NS = 16
NW = NC * NS
TOK_W = TOTAL // NW
GCH = 128
NCH = TOK_W // GCH
NV = TOK_W // L


def _body(vec_hbm, seg_hbm, ftab_hbm, stab_hbm, part_hbm,
          vec_v, fidx_v, sidx_v, seg_v, fval_v, sval_v,
          acc_a, acc_w, pad_v, shared, red_v, sem):
    cid = lax.axis_index("c")
    sid = lax.axis_index("s")
    wid = cid * NS + sid
    base = pl.multiple_of(wid * TOK_W, TOK_W)

    cp1 = pltpu.async_copy(vec_hbm.at[pl.ds(2 * base, 2 * TOK_W)], vec_v, sem)
    cp3 = pltpu.async_copy(seg_hbm.at[pl.ds(base, TOK_W)], seg_v, sem)
    cp1.wait()
    cp3.wait()

    iota2 = lax.iota(jnp.int32, L) * 2
    for c in range(NV):
        p = iota2 + (2 * L * c)
        sidx_v[pl.ds(c * L, L)] = plsc.load_gather(vec_v, [p])
        fidx_v[pl.ds(c * L, L)] = plsc.load_gather(vec_v, [p + 1])

    gf = pltpu.async_copy(ftab_hbm.at[fidx_v], fval_v, sem)
    gs = pltpu.async_copy(stab_hbm.at[sidx_v], sval_v, sem)

    acc_a[...] = jnp.zeros((L,), jnp.float32)
    acc_w[...] = jnp.zeros((L,), jnp.float32)
    gf.wait()
    gs.wait()
    for c in range(NV):
        sl = pl.ds(c * L, L)
        attn = jnp.exp(fval_v[sl])
        w = attn * sval_v[sl]
        seg = seg_v[sl]
        plsc.addupdate_scatter(acc_a, [seg], attn)
        plsc.addupdate_scatter(acc_w, [seg], w)

    pad_v[pl.ds(0, L)] = acc_a[...]
    pad_v[pl.ds(L, L)] = acc_w[...]
    pltpu.sync_copy(pad_v, shared.at[sid])
    plsc.subcore_barrier()

    @pl.when(sid == 0)
    def _():
        pltpu.sync_copy(shared, red_v)
        ta = red_v[0, pl.ds(0, L)]
        tw = red_v[0, pl.ds(L, L)]
        for s in range(1, NS):
            ta = ta + red_v[s, pl.ds(0, L)]
            tw = tw + red_v[s, pl.ds(L, L)]
        pad_v[pl.ds(0, L)] = ta
        pad_v[pl.ds(L, L)] = tw
        pltpu.sync_copy(pad_v, part_hbm.at[cid])


@functools.lru_cache(maxsize=1)
def _make_fragnet():
    return functools.partial(
        pl.kernel,
        mesh=plsc.VectorSubcoreMesh(core_axis_name="c", subcore_axis_name="s",
                                    num_cores=NC),
        out_type=jax.ShapeDtypeStruct((NC, 128), jnp.float32),
        compiler_params=pltpu.CompilerParams(
            needs_layout_passes=False,
            skip_device_barrier=True,
            disable_bounds_checks=True,
            disable_semaphore_checks=True,
        ),
        scratch_types=[
            pltpu.VMEM((2 * TOK_W,), jnp.int32),
            pltpu.VMEM((TOK_W,), jnp.int32),
            pltpu.VMEM((TOK_W,), jnp.int32),
            pltpu.VMEM((TOK_W,), jnp.int32),
            pltpu.VMEM((TOK_W,), jnp.float32),
            pltpu.VMEM((TOK_W,), jnp.float32),
            pltpu.VMEM((L,), jnp.float32),
            pltpu.VMEM((L,), jnp.float32),
            pltpu.VMEM((128,), jnp.float32),
            pltpu.VMEM_SHARED((NS, 128), jnp.float32),
            pltpu.VMEM((NS, 128), jnp.float32),
            pltpu.SemaphoreType.DMA,
        ],
    )(_body)


def _combine_body(part_ref, bias_ref, out_ref):
    pa = part_ref[0, pl.ds(0, L)] + part_ref[1, pl.ds(0, L)]
    pw = part_ref[0, pl.ds(L, L)] + part_ref[1, pl.ds(L, L)]
    out_ref[...] = pw / (pa + jnp.float32(0.001)) + bias_ref[0]


def _combine(partials, bias):
    return pl.pallas_call(
        _combine_body,
        out_shape=jax.ShapeDtypeStruct((NSEG,), jnp.float32),
    )(partials, bias)


def kernel(vectors, segment_ids, frag_table, site_table, bias):
    ftab = jnp.reshape(frag_table, (-1,))
    stab = jnp.reshape(site_table, (-1,))
    vec_flat = jnp.reshape(vectors, (-1,))
    partials = _make_fragnet()(vec_flat, segment_ids, ftab, stab)
    return _combine(partials, bias)


# speedup vs baseline: 1.6505x; 1.6505x over previous
import functools

import jax
import jax.numpy as jnp
from jax import lax
from jax.experimental import pallas as pl
from jax.experimental.pallas import tpu as pltpu
from jax.experimental.pallas import tpu_sc as plsc

TOTAL = 32768
NSEG = 16
L = 16
NC = 2
NS = 16
NW = NC * NS
TOK_W = TOTAL // NW
GCH = 128
NCH = TOK_W // GCH
NV = TOK_W // L


def _body(fidx_hbm, sidx_hbm, seg_hbm, ftab_hbm, stab_hbm, part_hbm,
          fidx_v, sidx_v, seg_v, fval_v, sval_v,
          acc_a, acc_w, pad_v, shared, red_v, sem):
    cid = lax.axis_index("c")
    sid = lax.axis_index("s")
    wid = cid * NS + sid
    base = pl.multiple_of(wid * TOK_W, TOK_W)

    cp1 = pltpu.async_copy(fidx_hbm.at[pl.ds(base, TOK_W)], fidx_v, sem)
    cp2 = pltpu.async_copy(sidx_hbm.at[pl.ds(base, TOK_W)], sidx_v, sem)
    cp3 = pltpu.async_copy(seg_hbm.at[pl.ds(base, TOK_W)], seg_v, sem)

    H = TOK_W // 2
    lo, hi = pl.ds(0, H), pl.ds(H, H)
    cp1.wait()
    gf0 = pltpu.async_copy(ftab_hbm.at[fidx_v.at[lo]], fval_v.at[lo], sem)
    gf1 = pltpu.async_copy(ftab_hbm.at[fidx_v.at[hi]], fval_v.at[hi], sem)
    cp2.wait()
    gs0 = pltpu.async_copy(stab_hbm.at[sidx_v.at[lo]], sval_v.at[lo], sem)
    gs1 = pltpu.async_copy(stab_hbm.at[sidx_v.at[hi]], sval_v.at[hi], sem)

    acc_a[...] = jnp.zeros((L,), jnp.float32)
    acc_w[...] = jnp.zeros((L,), jnp.float32)
    cp3.wait()
    gf0.wait()
    gs0.wait()
    for c in range(NV):
        if c == NV // 2:
            gf1.wait()
            gs1.wait()
        sl = pl.ds(c * L, L)
        attn = jnp.exp(fval_v[sl])
        w = attn * sval_v[sl]
        seg = seg_v[sl]
        plsc.addupdate_scatter(acc_a, [seg], attn)
        plsc.addupdate_scatter(acc_w, [seg], w)

    pad_v[pl.ds(0, L)] = acc_a[...]
    pad_v[pl.ds(L, L)] = acc_w[...]
    pltpu.sync_copy(pad_v, shared.at[sid])
    plsc.subcore_barrier()

    @pl.when(sid == 0)
    def _():
        pltpu.sync_copy(shared, red_v)
        ta = red_v[0, pl.ds(0, L)]
        tw = red_v[0, pl.ds(L, L)]
        for s in range(1, NS):
            ta = ta + red_v[s, pl.ds(0, L)]
            tw = tw + red_v[s, pl.ds(L, L)]
        pad_v[pl.ds(0, L)] = ta
        pad_v[pl.ds(L, L)] = tw
        pltpu.sync_copy(pad_v, part_hbm.at[cid])


@functools.lru_cache(maxsize=1)
def _make_fragnet():
    return functools.partial(
        pl.kernel,
        mesh=plsc.VectorSubcoreMesh(core_axis_name="c", subcore_axis_name="s",
                                    num_cores=NC),
        out_type=jax.ShapeDtypeStruct((NC, 128), jnp.float32),
        compiler_params=pltpu.CompilerParams(
            needs_layout_passes=False,
            skip_device_barrier=True,
            disable_bounds_checks=True,
            disable_semaphore_checks=True,
        ),
        scratch_types=[
            pltpu.VMEM((TOK_W,), jnp.int32),
            pltpu.VMEM((TOK_W,), jnp.int32),
            pltpu.VMEM((TOK_W,), jnp.int32),
            pltpu.VMEM((TOK_W,), jnp.float32),
            pltpu.VMEM((TOK_W,), jnp.float32),
            pltpu.VMEM((L,), jnp.float32),
            pltpu.VMEM((L,), jnp.float32),
            pltpu.VMEM((128,), jnp.float32),
            pltpu.VMEM_SHARED((NS, 128), jnp.float32),
            pltpu.VMEM((NS, 128), jnp.float32),
            pltpu.SemaphoreType.DMA,
        ],
    )(_body)


def _combine_body(part_ref, bias_ref, out_ref):
    pa = part_ref[0, pl.ds(0, L)] + part_ref[1, pl.ds(0, L)]
    pw = part_ref[0, pl.ds(L, L)] + part_ref[1, pl.ds(L, L)]
    out_ref[...] = pw / (pa + jnp.float32(0.001)) + bias_ref[0]


def _combine(partials, bias):
    return pl.pallas_call(
        _combine_body,
        out_shape=jax.ShapeDtypeStruct((NSEG,), jnp.float32),
    )(partials, bias)


def kernel(vectors, segment_ids, frag_table, site_table, bias):
    fidx = vectors[:, 1]
    sidx = vectors[:, 0]
    ftab = jnp.reshape(frag_table, (-1,))
    stab = jnp.reshape(site_table, (-1,))
    partials = _make_fragnet()(fidx, sidx, segment_ids, ftab, stab)
    return _combine(partials, bias)


# speedup vs baseline: 1.6546x; 1.0025x over previous
import functools

import jax
import jax.numpy as jnp
from jax import lax
from jax.experimental import pallas as pl
from jax.experimental.pallas import tpu as pltpu
from jax.experimental.pallas import tpu_sc as plsc

TOTAL = 32768
NSEG = 16
L = 16
NC = 2
NS = 16
NW = NC * NS
TOK_W = TOTAL // NW
GCH = 128
NCH = TOK_W // GCH
NV = TOK_W // L


def _body(fidx_hbm, sidx_hbm, seg_hbm, ftab_hbm, stab_hbm, part_hbm,
          fidx_v, sidx_v, seg_v, fval_v, sval_v,
          acc_a, acc_w, pad_v, shared, red_v, sem):
    cid = lax.axis_index("c")
    sid = lax.axis_index("s")
    wid = cid * NS + sid
    base = pl.multiple_of(wid * TOK_W, TOK_W)

    cp1 = pltpu.async_copy(fidx_hbm.at[pl.ds(base, TOK_W)], fidx_v, sem)
    cp2 = pltpu.async_copy(sidx_hbm.at[pl.ds(base, TOK_W)], sidx_v, sem)
    cp3 = pltpu.async_copy(seg_hbm.at[pl.ds(base, TOK_W)], seg_v, sem)

    cp1.wait()
    gf = pltpu.async_copy(ftab_hbm.at[fidx_v], fval_v, sem)
    cp2.wait()
    gs = pltpu.async_copy(stab_hbm.at[sidx_v], sval_v, sem)

    acc_a[...] = jnp.zeros((L,), jnp.float32)
    acc_w[...] = jnp.zeros((L,), jnp.float32)
    cp3.wait()
    gf.wait()
    gs.wait()
    for c in range(NV):
        sl = pl.ds(c * L, L)
        attn = jnp.exp(fval_v[sl])
        w = attn * sval_v[sl]
        seg = seg_v[sl]
        plsc.addupdate_scatter(acc_a, [seg], attn)
        plsc.addupdate_scatter(acc_w, [seg], w)

    pad_v[pl.ds(0, L)] = acc_a[...]
    pad_v[pl.ds(L, L)] = acc_w[...]
    pltpu.sync_copy(pad_v, shared.at[sid])
    plsc.subcore_barrier()

    @pl.when(sid == 0)
    def _():
        pltpu.sync_copy(shared, red_v)
        ta = red_v[0, pl.ds(0, L)]
        tw = red_v[0, pl.ds(L, L)]
        for s in range(1, NS):
            ta = ta + red_v[s, pl.ds(0, L)]
            tw = tw + red_v[s, pl.ds(L, L)]
        pad_v[pl.ds(0, L)] = ta
        pad_v[pl.ds(L, L)] = tw
        pltpu.sync_copy(pad_v, part_hbm.at[cid])


@functools.lru_cache(maxsize=1)
def _make_fragnet():
    return functools.partial(
        pl.kernel,
        mesh=plsc.VectorSubcoreMesh(core_axis_name="c", subcore_axis_name="s",
                                    num_cores=NC),
        out_type=jax.ShapeDtypeStruct((NC, 128), jnp.float32),
        compiler_params=pltpu.CompilerParams(
            needs_layout_passes=False,
            skip_device_barrier=True,
            disable_bounds_checks=True,
            disable_semaphore_checks=True,
        ),
        scratch_types=[
            pltpu.VMEM((TOK_W,), jnp.int32),
            pltpu.VMEM((TOK_W,), jnp.int32),
            pltpu.VMEM((TOK_W,), jnp.int32),
            pltpu.VMEM((TOK_W,), jnp.float32),
            pltpu.VMEM((TOK_W,), jnp.float32),
            pltpu.VMEM((L,), jnp.float32),
            pltpu.VMEM((L,), jnp.float32),
            pltpu.VMEM((128,), jnp.float32),
            pltpu.VMEM_SHARED((NS, 128), jnp.float32),
            pltpu.VMEM((NS, 128), jnp.float32),
            pltpu.SemaphoreType.DMA,
        ],
    )(_body)


def _combine_body(part_ref, bias_ref, out_ref):
    pa = part_ref[0, pl.ds(0, L)] + part_ref[1, pl.ds(0, L)]
    pw = part_ref[0, pl.ds(L, L)] + part_ref[1, pl.ds(L, L)]
    out_ref[...] = pw / (pa + jnp.float32(0.001)) + bias_ref[0]


def _combine(partials, bias):
    return pl.pallas_call(
        _combine_body,
        out_shape=jax.ShapeDtypeStruct((NSEG,), jnp.float32),
    )(partials, bias)


def kernel(vectors, segment_ids, frag_table, site_table, bias):
    fidx = vectors[:, 1]
    sidx = vectors[:, 0]
    ftab = jnp.reshape(frag_table, (-1,))
    stab = jnp.reshape(site_table, (-1,))
    partials = _make_fragnet()(fidx, sidx, segment_ids, ftab, stab)
    return _combine(partials, bias)


# speedup vs baseline: 1.6637x; 1.0055x over previous
import functools

import jax
import jax.numpy as jnp
from jax import lax
from jax.experimental import pallas as pl
from jax.experimental.pallas import tpu as pltpu
from jax.experimental.pallas import tpu_sc as plsc

TOTAL = 32768
NSEG = 16
L = 16
NC = 2
NS = 16
NW = NC * NS
TOK_W = TOTAL // NW
GCH = 128
NCH = TOK_W // GCH
NV = TOK_W // L


def _body(fidx_hbm, sidx_hbm, seg_hbm, ftab_hbm, stab_hbm, part_hbm,
          fidx_v, sidx_v, seg_v, fval_v, sval_v,
          acc_a, acc_w, pad_v, sem):
    cid = lax.axis_index("c")
    sid = lax.axis_index("s")
    wid = cid * NS + sid
    base = pl.multiple_of(wid * TOK_W, TOK_W)

    cp1 = pltpu.async_copy(fidx_hbm.at[pl.ds(base, TOK_W)], fidx_v, sem)
    cp2 = pltpu.async_copy(sidx_hbm.at[pl.ds(base, TOK_W)], sidx_v, sem)
    cp3 = pltpu.async_copy(seg_hbm.at[pl.ds(base, TOK_W)], seg_v, sem)

    cp1.wait()
    gf = pltpu.async_copy(ftab_hbm.at[fidx_v], fval_v, sem)
    cp2.wait()
    gs = pltpu.async_copy(stab_hbm.at[sidx_v], sval_v, sem)

    acc_a[...] = jnp.zeros((L,), jnp.float32)
    acc_w[...] = jnp.zeros((L,), jnp.float32)
    cp3.wait()
    gf.wait()
    gs.wait()
    for c in range(NV):
        sl = pl.ds(c * L, L)
        attn = jnp.exp(fval_v[sl])
        w = attn * sval_v[sl]
        seg = seg_v[sl]
        plsc.addupdate_scatter(acc_a, [seg], attn)
        plsc.addupdate_scatter(acc_w, [seg], w)

    pad_v[pl.ds(0, L)] = acc_a[...]
    pad_v[pl.ds(L, L)] = acc_w[...]
    pltpu.sync_copy(pad_v, part_hbm.at[wid])


@functools.lru_cache(maxsize=1)
def _make_fragnet():
    return functools.partial(
        pl.kernel,
        mesh=plsc.VectorSubcoreMesh(core_axis_name="c", subcore_axis_name="s",
                                    num_cores=NC),
        out_type=jax.ShapeDtypeStruct((NW, 128), jnp.float32),
        compiler_params=pltpu.CompilerParams(
            needs_layout_passes=False,
            skip_device_barrier=True,
            disable_bounds_checks=True,
            disable_semaphore_checks=True,
        ),
        scratch_types=[
            pltpu.VMEM((TOK_W,), jnp.int32),
            pltpu.VMEM((TOK_W,), jnp.int32),
            pltpu.VMEM((TOK_W,), jnp.int32),
            pltpu.VMEM((TOK_W,), jnp.float32),
            pltpu.VMEM((TOK_W,), jnp.float32),
            pltpu.VMEM((L,), jnp.float32),
            pltpu.VMEM((L,), jnp.float32),
            pltpu.VMEM((128,), jnp.float32),
            pltpu.SemaphoreType.DMA,
        ],
    )(_body)


def _combine_body(part_ref, bias_ref, out_ref):
    pa = jnp.sum(part_ref[:, :L], axis=0)
    pw = jnp.sum(part_ref[:, L:2 * L], axis=0)
    out_ref[...] = pw / (pa + jnp.float32(0.001)) + bias_ref[0]


def _combine(partials, bias):
    return pl.pallas_call(
        _combine_body,
        out_shape=jax.ShapeDtypeStruct((NSEG,), jnp.float32),
    )(partials, bias)


def kernel(vectors, segment_ids, frag_table, site_table, bias):
    fidx = vectors[:, 1]
    sidx = vectors[:, 0]
    ftab = jnp.reshape(frag_table, (-1,))
    stab = jnp.reshape(site_table, (-1,))
    partials = _make_fragnet()(fidx, sidx, segment_ids, ftab, stab)
    return _combine(partials, bias)


# speedup vs baseline: 1.7278x; 1.0385x over previous
import functools

import jax
import jax.numpy as jnp
from jax import lax
from jax.experimental import pallas as pl
from jax.experimental.pallas import tpu as pltpu
from jax.experimental.pallas import tpu_sc as plsc

TOTAL = 32768
NSEG = 16
L = 16
NC = 2
NS = 16
NW = NC * NS
TOK_W = TOTAL // NW
GCH = 128
NCH = TOK_W // GCH
NV = TOK_W // L


def _body(fidx_hbm, sidx_hbm, seg_hbm, ftab_hbm, stab_hbm, part_hbm,
          fidx_v, sidx_v, seg_v, fval_v, sval_v,
          acc_a, acc_w, pad_v, sem):
    cid = lax.axis_index("c")
    sid = lax.axis_index("s")
    wid = cid * NS + sid
    base = pl.multiple_of(wid * TOK_W, TOK_W)

    cp1 = pltpu.async_copy(fidx_hbm.at[pl.ds(base, TOK_W)], fidx_v, sem)
    cp2 = pltpu.async_copy(sidx_hbm.at[pl.ds(base, TOK_W)], sidx_v, sem)
    cp3 = pltpu.async_copy(seg_hbm.at[pl.ds(base, TOK_W)], seg_v, sem)

    cp1.wait()
    gf = pltpu.async_copy(ftab_hbm.at[fidx_v], fval_v, sem)
    cp2.wait()
    gs = pltpu.async_copy(stab_hbm.at[sidx_v], sval_v, sem)

    acc_a[...] = jnp.zeros((L,), jnp.float32)
    acc_w[...] = jnp.zeros((L,), jnp.float32)
    cp3.wait()
    gf.wait()
    gs.wait()
    lane = lax.iota(jnp.int32, L)
    for c in range(NV):
        sl = pl.ds(c * L, L)
        attn = jnp.exp(fval_v[sl])
        w = attn * sval_v[sl]
        seg = seg_v[sl]
        seg0 = seg[0]
        uniform = seg0 == seg[L - 1]

        @pl.when(uniform)
        def _():
            sa = jnp.sum(attn)
            sw = jnp.sum(w)
            hot = lane == seg0
            acc_a[...] = acc_a[...] + jnp.where(hot, sa, jnp.float32(0))
            acc_w[...] = acc_w[...] + jnp.where(hot, sw, jnp.float32(0))

        @pl.when(jnp.logical_not(uniform))
        def _():
            plsc.addupdate_scatter(acc_a, [seg], attn)
            plsc.addupdate_scatter(acc_w, [seg], w)

    pad_v[pl.ds(0, L)] = acc_a[...]
    pad_v[pl.ds(L, L)] = acc_w[...]
    pltpu.sync_copy(pad_v, part_hbm.at[wid])


@functools.lru_cache(maxsize=1)
def _make_fragnet():
    return functools.partial(
        pl.kernel,
        mesh=plsc.VectorSubcoreMesh(core_axis_name="c", subcore_axis_name="s",
                                    num_cores=NC),
        out_type=jax.ShapeDtypeStruct((NW, 128), jnp.float32),
        compiler_params=pltpu.CompilerParams(
            needs_layout_passes=False,
            skip_device_barrier=True,
            disable_bounds_checks=True,
            disable_semaphore_checks=True,
        ),
        scratch_types=[
            pltpu.VMEM((TOK_W,), jnp.int32),
            pltpu.VMEM((TOK_W,), jnp.int32),
            pltpu.VMEM((TOK_W,), jnp.int32),
            pltpu.VMEM((TOK_W,), jnp.float32),
            pltpu.VMEM((TOK_W,), jnp.float32),
            pltpu.VMEM((L,), jnp.float32),
            pltpu.VMEM((L,), jnp.float32),
            pltpu.VMEM((128,), jnp.float32),
            pltpu.SemaphoreType.DMA,
        ],
    )(_body)


def _combine_body(part_ref, bias_ref, out_ref):
    pa = jnp.sum(part_ref[:, :L], axis=0)
    pw = jnp.sum(part_ref[:, L:2 * L], axis=0)
    out_ref[...] = pw / (pa + jnp.float32(0.001)) + bias_ref[0]


def _combine(partials, bias):
    return pl.pallas_call(
        _combine_body,
        out_shape=jax.ShapeDtypeStruct((NSEG,), jnp.float32),
    )(partials, bias)


def kernel(vectors, segment_ids, frag_table, site_table, bias):
    fidx = vectors[:, 1]
    sidx = vectors[:, 0]
    ftab = jnp.reshape(frag_table, (-1,))
    stab = jnp.reshape(site_table, (-1,))
    partials = _make_fragnet()(fidx, sidx, segment_ids, ftab, stab)
    return _combine(partials, bias)


# speedup vs baseline: 1.7790x; 1.0297x over previous
import functools

import jax
import jax.numpy as jnp
from jax import lax
from jax.experimental import pallas as pl
from jax.experimental.pallas import tpu as pltpu
from jax.experimental.pallas import tpu_sc as plsc

TOTAL = 32768
NSEG = 16
L = 16
NC = 2
NS = 16
NW = NC * NS
TOK_W = TOTAL // NW
GCH = 128
NCH = TOK_W // GCH
NV = TOK_W // L


def _body(fidx_hbm, sidx_hbm, seg_hbm, ftab_hbm, stab_hbm, part_hbm,
          fidx_v, sidx_v, seg_v, fval_v, sval_v,
          acc_a, acc_w, pad_v, sem, semf0, sems0, semf1, sems1):
    cid = lax.axis_index("c")
    sid = lax.axis_index("s")
    wid = cid * NS + sid
    base = pl.multiple_of(wid * TOK_W, TOK_W)

    cp1 = pltpu.async_copy(fidx_hbm.at[pl.ds(base, TOK_W)], fidx_v, sem)
    cp2 = pltpu.async_copy(sidx_hbm.at[pl.ds(base, TOK_W)], sidx_v, sem)
    cp3 = pltpu.async_copy(seg_hbm.at[pl.ds(base, TOK_W)], seg_v, sem)

    H = TOK_W // 2
    lo, hi = pl.ds(0, H), pl.ds(H, H)
    cp1.wait()
    gf0 = pltpu.async_copy(ftab_hbm.at[fidx_v.at[lo]], fval_v.at[lo], semf0)
    gf1 = pltpu.async_copy(ftab_hbm.at[fidx_v.at[hi]], fval_v.at[hi], semf1)
    cp2.wait()
    gs0 = pltpu.async_copy(stab_hbm.at[sidx_v.at[lo]], sval_v.at[lo], sems0)
    gs1 = pltpu.async_copy(stab_hbm.at[sidx_v.at[hi]], sval_v.at[hi], sems1)

    acc_a[...] = jnp.zeros((L,), jnp.float32)
    acc_w[...] = jnp.zeros((L,), jnp.float32)
    cp3.wait()
    gf0.wait()
    gs0.wait()
    lane = lax.iota(jnp.int32, L)
    for c in range(NV):
        if c == NV // 2:
            gf1.wait()
            gs1.wait()
        sl = pl.ds(c * L, L)
        attn = jnp.exp(fval_v[sl])
        w = attn * sval_v[sl]
        seg = seg_v[sl]
        seg0 = seg[0]
        uniform = seg0 == seg[L - 1]

        @pl.when(uniform)
        def _():
            sa = jnp.sum(attn)
            sw = jnp.sum(w)
            hot = lane == seg0
            acc_a[...] = acc_a[...] + jnp.where(hot, sa, jnp.float32(0))
            acc_w[...] = acc_w[...] + jnp.where(hot, sw, jnp.float32(0))

        @pl.when(jnp.logical_not(uniform))
        def _():
            plsc.addupdate_scatter(acc_a, [seg], attn)
            plsc.addupdate_scatter(acc_w, [seg], w)

    pad_v[pl.ds(0, L)] = acc_a[...]
    pad_v[pl.ds(L, L)] = acc_w[...]
    pltpu.sync_copy(pad_v, part_hbm.at[wid])


@functools.lru_cache(maxsize=1)
def _make_fragnet():
    return functools.partial(
        pl.kernel,
        mesh=plsc.VectorSubcoreMesh(core_axis_name="c", subcore_axis_name="s",
                                    num_cores=NC),
        out_type=jax.ShapeDtypeStruct((NW, 128), jnp.float32),
        compiler_params=pltpu.CompilerParams(
            needs_layout_passes=False,
            skip_device_barrier=True,
            disable_bounds_checks=True,
            disable_semaphore_checks=True,
        ),
        scratch_types=[
            pltpu.VMEM((TOK_W,), jnp.int32),
            pltpu.VMEM((TOK_W,), jnp.int32),
            pltpu.VMEM((TOK_W,), jnp.int32),
            pltpu.VMEM((TOK_W,), jnp.float32),
            pltpu.VMEM((TOK_W,), jnp.float32),
            pltpu.VMEM((L,), jnp.float32),
            pltpu.VMEM((L,), jnp.float32),
            pltpu.VMEM((128,), jnp.float32),
            pltpu.SemaphoreType.DMA,
            pltpu.SemaphoreType.DMA,
            pltpu.SemaphoreType.DMA,
            pltpu.SemaphoreType.DMA,
            pltpu.SemaphoreType.DMA,
        ],
    )(_body)


def _combine_body(part_ref, bias_ref, out_ref):
    pa = jnp.sum(part_ref[:, :L], axis=0)
    pw = jnp.sum(part_ref[:, L:2 * L], axis=0)
    out_ref[...] = pw / (pa + jnp.float32(0.001)) + bias_ref[0]


def _combine(partials, bias):
    return pl.pallas_call(
        _combine_body,
        out_shape=jax.ShapeDtypeStruct((NSEG,), jnp.float32),
    )(partials, bias)


def kernel(vectors, segment_ids, frag_table, site_table, bias):
    fidx = vectors[:, 1]
    sidx = vectors[:, 0]
    ftab = jnp.reshape(frag_table, (-1,))
    stab = jnp.reshape(site_table, (-1,))
    partials = _make_fragnet()(fidx, sidx, segment_ids, ftab, stab)
    return _combine(partials, bias)
